# baseline (speedup 1.0000x reference)
import functools

import jax
import jax.numpy as jnp
from jax import lax
from jax.experimental import pallas as pl
from jax.experimental.pallas import tpu as pltpu
from jax.experimental.pallas import tpu_sc as plsc

CLASSES_ = 100000
SMOOTH_ = 0.1
CONF_ = 1.0 - SMOOTH_
ROWS_ = 1024
CB_ = 2048

_CT_ = 79872
_SCK_ = 2048
_NSCK_ = 9
_TAIL0_ = _CT_ + _NSCK_ * _SCK_

NTC_ = _CT_ // CB_
NCHUNK_ = NTC_ + 1
NPS_ = -(-ROWS_ // NCHUNK_)

_NW_ = 16
_RPW_ = ROWS_ // _NW_


def _sc_partials(pred):
    mesh = plsc.VectorSubcoreMesh(core_axis_name="c", subcore_axis_name="s", num_cores=1)

    @functools.partial(
        pl.kernel,
        mesh=mesh,
        out_type=[jax.ShapeDtypeStruct((ROWS_, 128), jnp.float32)] * 3,
        scratch_types=[
            pltpu.VMEM((8, _SCK_), jnp.float32),
            pltpu.VMEM((8, 128), jnp.float32),
            pltpu.VMEM((8, 128), jnp.float32),
            pltpu.VMEM((8, 128), jnp.float32),
        ],
    )
    def sc_kernel(pred_hbm, m_hbm, s_hbm, p_hbm, buf, mst, sst, pst):
        wid = lax.axis_index("s")
        base = wid * _RPW_
        ninf = jnp.full((16,), -jnp.inf, jnp.float32)
        zero = jnp.zeros((16,), jnp.float32)
        for tr in range(_RPW_ // 8):
            r0 = base + tr * 8
            def chunk_body(c, carry):
                col0 = pl.multiple_of(_CT_ + c * _SCK_, 128)
                pltpu.sync_copy(
                    pred_hbm.at[pl.ds(r0, 8), pl.ds(col0, _SCK_)], buf
                )
                out = []
                for row in range(8):
                    m16, s16, p16 = carry[row], carry[8 + row], carry[16 + row]

                    def grp_body(i, acc):
                        m, s, p = acc
                        v = buf[row, pl.ds(pl.multiple_of(i * 16, 16), 16)]
                        mn = jnp.maximum(m, v)
                        s = s * jnp.exp(m - mn) + jnp.exp(v - mn)
                        return mn, s, p + v

                    out.append(
                        lax.fori_loop(0, _SCK_ // 16, grp_body, (m16, s16, p16))
                    )
                return (
                    tuple(o[0] for o in out)
                    + tuple(o[1] for o in out)
                    + tuple(o[2] for o in out)
                )

            accs = lax.fori_loop(
                0, _NSCK_, chunk_body, (ninf,) * 8 + (zero,) * 16
            )
            for row in range(8):
                mst[row, pl.ds(0, 16)] = accs[row]
                sst[row, pl.ds(0, 16)] = accs[8 + row]
                pst[row, pl.ds(0, 16)] = accs[16 + row]
            pltpu.sync_copy(mst, m_hbm.at[pl.ds(r0, 8), :])
            pltpu.sync_copy(sst, s_hbm.at[pl.ds(r0, 8), :])
            pltpu.sync_copy(pst, p_hbm.at[pl.ds(r0, 8), :])

    return sc_kernel(pred)


def _tc_kernel(tgt_sref, *refs):
    x_ref = refs[0]
    gblocks = refs[1 : 1 + NPS_]
    m_ref, s_ref, p_ref, tval_ref = refs[1 + NPS_ : 5 + NPS_]
    tsel_ref = refs[5 + NPS_]
    j = pl.program_id(0)

    @pl.when(j == 0)
    def _init():
        m_ref[...] = jnp.full_like(m_ref, -jnp.inf)
        s_ref[...] = jnp.zeros_like(s_ref)
        p_ref[...] = jnp.zeros_like(p_ref)
        tsel_ref[...] = jnp.zeros_like(tsel_ref)

    sub = lax.broadcasted_iota(jnp.int32, (8, 1), 0)
    lane = lax.broadcasted_iota(jnp.int32, (8, 128), 1)
    for c in range(NPS_):
        fraw = j * NPS_ + c
        f = jnp.minimum(fraw, ROWS_ - 1)
        hit = (sub == f % 8) & (lane == tgt_sref[f] % 128) & (fraw < ROWS_)
        row0 = pl.multiple_of((f // 8) * 8, 8)
        tsel_ref[pl.ds(row0, 8), :] += jnp.where(hit, gblocks[c][...], 0.0)

    def _step(masked):
        x = x_ref[...]
        if masked:
            col = lax.broadcasted_iota(jnp.int32, x.shape, 1)
            valid = col < (CLASSES_ - _TAIL0_)
            xm = jnp.where(valid, x, -jnp.inf)
            xs = jnp.where(valid, x, 0.0)
        else:
            xm = x
            xs = x
        m_old = m_ref[...]
        mc = jnp.max(xm, axis=-1, keepdims=True)
        m_new = jnp.maximum(m_old, mc)
        e = jnp.exp(xm - m_new)
        s_ref[...] = s_ref[...] * jnp.exp(m_old - m_new) + jnp.sum(
            e, axis=-1, keepdims=True
        )
        m_ref[...] = m_new
        p_ref[...] = p_ref[...] + jnp.sum(xs, axis=-1, keepdims=True)

    pl.when(j < NTC_)(lambda: _step(False))
    pl.when(j >= NTC_)(lambda: _step(True))

    @pl.when(j == NCHUNK_ - 1)
    def _fini():
        tval_ref[...] = jnp.sum(tsel_ref[...], axis=-1, keepdims=True)


def _combine_kernel(mt_ref, st_ref, pt_ref, tv_ref, ms_ref, ss_ref, ps_ref, out_ref):
    lane = lax.broadcasted_iota(jnp.int32, (ROWS_, 128), 1)
    ok = lane < 16
    ms = jnp.where(ok, ms_ref[...], -jnp.inf)
    ss = jnp.where(ok, ss_ref[...], 0.0)
    ps = jnp.where(ok, ps_ref[...], 0.0)
    mt = mt_ref[...]
    m_all = jnp.maximum(mt, jnp.max(ms, axis=-1, keepdims=True))
    s_all = st_ref[...] * jnp.exp(mt - m_all) + jnp.sum(
        ss * jnp.exp(ms - m_all), axis=-1, keepdims=True
    )
    p_all = pt_ref[...] + jnp.sum(ps, axis=-1, keepdims=True)
    lse = m_all + jnp.log(s_all)
    sum_logp = p_all - CLASSES_ * lse
    t_logp = tv_ref[...] - lse
    loss = -(SMOOTH_ / (CLASSES_ - 1)) * (sum_logp - t_logp) - CONF_ * t_logp
    out_ref[...] = (jnp.sum(loss) / ROWS_).reshape(1, 1)


def _gspec(c):
    def idx(j, tgt):
        f = jnp.minimum(j * NPS_ + c, ROWS_ - 1)
        return (f // 8, tgt[f] // 128)

    return pl.BlockSpec((8, 128), idx)


@jax.jit
def kernel(pred, target):
    tgt = target.astype(jnp.int32)
    part = pl.pallas_call(
        _tc_kernel,
        grid_spec=pltpu.PrefetchScalarGridSpec(
            num_scalar_prefetch=1,
            grid=(NCHUNK_,),
            in_specs=[
                pl.BlockSpec(
                    (ROWS_, CB_),
                    lambda j, tgt: (0, jnp.where(j == NTC_, _TAIL0_ // CB_, j)),
                )
            ]
            + [_gspec(c) for c in range(NPS_)],
            out_specs=[pl.BlockSpec((ROWS_, 1), lambda j, tgt: (0, 0))] * 4,
            scratch_shapes=[pltpu.VMEM((ROWS_, 128), jnp.float32)],
        ),
        out_shape=[jax.ShapeDtypeStruct((ROWS_, 1), jnp.float32)] * 4,
    )(tgt, *([pred] * (1 + NPS_)))
    m_sc, s_sc, p_sc = _sc_partials(pred)
    out = pl.pallas_call(
        _combine_kernel,
        out_shape=jax.ShapeDtypeStruct((1, 1), jnp.float32),
    )(*part, m_sc, s_sc, p_sc)
    return out[0, 0]


# speedup vs baseline: 1.4132x; 1.4132x over previous
import jax
import jax.numpy as jnp
from jax import lax
from jax.experimental import pallas as pl
from jax.experimental.pallas import tpu as pltpu

CLASSES_ = 100000
SMOOTH_ = 0.1
CONF_ = 1.0 - SMOOTH_
ROWS_ = 1024
CB_ = 2048
NFULL_ = CLASSES_ // CB_
NCHUNK_ = (CLASSES_ + CB_ - 1) // CB_
NPS_ = -(-ROWS_ // NCHUNK_)


def _loss_kernel(tgt_sref, *refs):
    x_ref = refs[0]
    gblocks = refs[1 : 1 + NPS_]
    out_ref = refs[1 + NPS_]
    m_ref, s_ref, p_ref, tsel_ref = refs[2 + NPS_ :]
    j = pl.program_id(0)

    @pl.when(j == 0)
    def _init():
        m_ref[...] = jnp.full_like(m_ref, -jnp.inf)
        s_ref[...] = jnp.zeros_like(s_ref)
        p_ref[...] = jnp.zeros_like(p_ref)
        tsel_ref[...] = jnp.zeros_like(tsel_ref)

    sub = lax.broadcasted_iota(jnp.int32, (8, 1), 0)
    lane = lax.broadcasted_iota(jnp.int32, (8, 128), 1)
    for c in range(NPS_):
        fraw = j * NPS_ + c
        f = jnp.minimum(fraw, ROWS_ - 1)
        hit = (sub == f % 8) & (lane == tgt_sref[f] % 128) & (fraw < ROWS_)
        row0 = pl.multiple_of((f // 8) * 8, 8)
        tsel_ref[pl.ds(row0, 8), :] += jnp.where(hit, gblocks[c][...], 0.0)

    def _step(masked):
        x = x_ref[...]
        if masked:
            col = lax.broadcasted_iota(jnp.int32, x.shape, 1)
            valid = col < (CLASSES_ - j * CB_)
            xm = jnp.where(valid, x, -jnp.inf)
            xs = jnp.where(valid, x, 0.0)
        else:
            xm = x
            xs = x
        m_old = m_ref[...]
        mc = jnp.max(xm, axis=-1, keepdims=True)
        m_new = jnp.maximum(m_old, mc)
        e = jnp.exp(xm - m_new)
        s_ref[...] = s_ref[...] * jnp.exp(m_old - m_new) + jnp.sum(
            e, axis=-1, keepdims=True
        )
        m_ref[...] = m_new
        p_ref[...] = p_ref[...] + jnp.sum(xs, axis=-1, keepdims=True)

    pl.when(j < NFULL_)(lambda: _step(False))
    pl.when(j >= NFULL_)(lambda: _step(True))

    @pl.when(j == NCHUNK_ - 1)
    def _fini():
        lse = m_ref[...] + jnp.log(s_ref[...])
        sum_logp = p_ref[...] - CLASSES_ * lse
        t_logp = jnp.sum(tsel_ref[...], axis=-1, keepdims=True) - lse
        loss = -(SMOOTH_ / (CLASSES_ - 1)) * (sum_logp - t_logp) - CONF_ * t_logp
        out_ref[...] = (jnp.sum(loss) / ROWS_).reshape(1, 1)


def _gspec(c):
    def idx(j, tgt):
        f = jnp.minimum(j * NPS_ + c, ROWS_ - 1)
        return (f // 8, tgt[f] // 128)

    return pl.BlockSpec((8, 128), idx)


@jax.jit
def kernel(pred, target):
    tgt = target.astype(jnp.int32)
    out = pl.pallas_call(
        _loss_kernel,
        grid_spec=pltpu.PrefetchScalarGridSpec(
            num_scalar_prefetch=1,
            grid=(NCHUNK_,),
            in_specs=[pl.BlockSpec((ROWS_, CB_), lambda j, tgt: (0, j))]
            + [_gspec(c) for c in range(NPS_)],
            out_specs=pl.BlockSpec((1, 1), lambda j, tgt: (0, 0)),
            scratch_shapes=[pltpu.VMEM((ROWS_, 1), jnp.float32)] * 3
            + [pltpu.VMEM((ROWS_, 128), jnp.float32)],
        ),
        out_shape=jax.ShapeDtypeStruct((1, 1), jnp.float32),
    )(tgt, *([pred] * (1 + NPS_)))
    return out[0, 0]
